# Initial kernel scaffold; baseline (speedup 1.0000x reference)
import jax
import jax.numpy as jnp
from jax.experimental import pallas as pl


def kernel(input_word, context_word, in_emb, out_emb, word_frequency):
    raise NotImplementedError("write your pallas kernel here")


# speedup vs baseline: 1317.1209x; 1317.1209x over previous
import functools

import jax
import jax.numpy as jnp
from jax import lax
from jax.experimental import pallas as pl
from jax.experimental.pallas import tpu as pltpu
from jax.experimental.pallas import tpu_sc as plsc

VOCAB = 1000000
EMB = 64
NEG = 5
BATCH = 16384

NW = 32
T = 4
S_TOTAL = BATCH * NEG
S_PER_W = S_TOTAL // NW
B_PER_W = BATCH // NW
CHUNK = 128
S_CHUNKS = S_PER_W // CHUNK
B_CHUNKS = B_PER_W // CHUNK


def _rng_body(prop_ref, u4_ref):
    pltpu.prng_seed(42)
    shape = prop_ref.shape
    b1 = pltpu.bitcast(pltpu.prng_random_bits(shape), jnp.uint32)
    b2 = pltpu.bitcast(pltpu.prng_random_bits(shape), jnp.uint32)
    prop_ref[...] = (b1 % jnp.uint32(VOCAB)).astype(jnp.int32)
    u = (b2 >> jnp.uint32(8)).astype(jnp.float32) * jnp.float32(2.0**-24)
    u2 = u * u
    u4_ref[...] = u2 * u2


def _make_rng():
    return pl.pallas_call(
        _rng_body,
        out_shape=(
            jax.ShapeDtypeStruct((NW * T, S_PER_W), jnp.int32),
            jax.ShapeDtypeStruct((NW * T, S_PER_W), jnp.float32),
        ),
    )


def _sc_body(wf_hbm, in_emb_hbm, out_emb_hbm, iw_hbm, cw_hbm, prop_hbm,
             u4_hbm, iv_hbm, ov_hbm, neg_hbm,
             prop_v, u4_v, wf_v, iwcw_v, negidx_v, buf_v, sem):
    wid = lax.axis_index("s") * 2 + lax.axis_index("c")

    pltpu.sync_copy(prop_hbm.at[pl.ds(wid * T, T)], prop_v)
    pltpu.sync_copy(u4_hbm.at[pl.ds(wid * T, T)], u4_v)
    pltpu.sync_copy(iw_hbm.at[pl.ds(wid * B_PER_W, B_PER_W)],
                    iwcw_v.at[0])
    pltpu.sync_copy(cw_hbm.at[pl.ds(wid * B_PER_W, B_PER_W)],
                    iwcw_v.at[1])

    for t in range(T):
        descs = []
        for c in range(S_CHUNKS):
            sl = pl.ds(c * CHUNK, CHUNK)
            descs.append(pltpu.async_copy(
                wf_hbm.at[prop_v.at[t, sl]], wf_v.at[t, sl], sem))
        for d in descs:
            d.wait()

    def accept(v, _):
        g = pl.ds(v * 16, 16)
        best = prop_v[T - 1, g]
        for t in reversed(range(T - 1)):
            w = wf_v[t, g]
            ok = u4_v[t, g] <= (w * w) * w
            best = jnp.where(ok, prop_v[t, g], best)
        negidx_v[g] = best
        return _

    lax.fori_loop(0, S_PER_W // 16, accept, 0)

    for c in range(S_CHUNKS):
        sl = pl.ds(c * CHUNK, CHUNK)
        pltpu.async_copy(out_emb_hbm.at[negidx_v.at[sl]], buf_v, sem).wait()
        pltpu.sync_copy(buf_v, neg_hbm.at[pl.ds(wid * S_PER_W + c * CHUNK,
                                                CHUNK)])

    for c in range(B_CHUNKS):
        sl = pl.ds(c * CHUNK, CHUNK)
        dst = pl.ds(wid * B_PER_W + c * CHUNK, CHUNK)
        pltpu.async_copy(in_emb_hbm.at[iwcw_v.at[0, sl]], buf_v, sem).wait()
        pltpu.sync_copy(buf_v, iv_hbm.at[dst])
        pltpu.async_copy(out_emb_hbm.at[iwcw_v.at[1, sl]], buf_v, sem).wait()
        pltpu.sync_copy(buf_v, ov_hbm.at[dst])


def _make_gather():
    mesh = plsc.VectorSubcoreMesh(core_axis_name="c", subcore_axis_name="s")
    return pl.kernel(
        _sc_body,
        mesh=mesh,
        compiler_params=pltpu.CompilerParams(use_tc_tiling_on_sc=False),
        out_type=(
            jax.ShapeDtypeStruct((BATCH, EMB), jnp.float32),
            jax.ShapeDtypeStruct((BATCH, EMB), jnp.float32),
            jax.ShapeDtypeStruct((S_TOTAL, EMB), jnp.float32),
        ),
        scratch_types=[
            pltpu.VMEM((T, S_PER_W), jnp.int32),
            pltpu.VMEM((T, S_PER_W), jnp.float32),
            pltpu.VMEM((T, S_PER_W), jnp.float32),
            pltpu.VMEM((2, B_PER_W), jnp.int32),
            pltpu.VMEM((S_PER_W,), jnp.int32),
            pltpu.VMEM((CHUNK, EMB), jnp.float32),
            pltpu.SemaphoreType.DMA,
        ],
    )


def _logsig(x):
    return jnp.minimum(x, 0.0) - jnp.log(1.0 + jnp.exp(-jnp.abs(x)))


def _loss_body(iv_ref, ov_ref, neg_ref, out_ref):
    i = pl.program_id(0)
    iv = iv_ref[...]
    pos = jnp.sum(_logsig(iv * ov_ref[...])) * (1.0 / EMB)
    scores = -jnp.sum(neg_ref[...] * iv[None, :, :], axis=2)
    neg = jnp.sum(_logsig(scores))
    @pl.when(i == 0)
    def _():
        out_ref[...] = jnp.zeros((1, 1), jnp.float32)
    out_ref[...] = out_ref[...] + (pos + neg)
    @pl.when(i == NW - 1)
    def _():
        out_ref[...] = out_ref[...] * jnp.float32(-1.0 / BATCH)


def _make_loss():
    return pl.pallas_call(
        _loss_body,
        grid=(NW,),
        in_specs=[
            pl.BlockSpec((B_PER_W, EMB), lambda i: (i, 0)),
            pl.BlockSpec((B_PER_W, EMB), lambda i: (i, 0)),
            pl.BlockSpec((NEG, B_PER_W, EMB), lambda i: (0, i, 0)),
        ],
        out_specs=pl.BlockSpec((1, 1), lambda i: (0, 0)),
        out_shape=jax.ShapeDtypeStruct((1, 1), jnp.float32),
    )


def kernel(input_word, context_word, in_emb, out_emb, word_frequency):
    iw = input_word.astype(jnp.int32)
    cw = context_word.astype(jnp.int32)
    prop, u4 = _make_rng()()
    iv, ov, neg = _make_gather()(word_frequency, in_emb, out_emb, iw, cw,
                                 prop, u4)
    neg3 = neg.reshape(NEG, BATCH, EMB)
    loss = _make_loss()(iv, ov, neg3)
    return loss.reshape(())


# speedup vs baseline: 1341.4822x; 1.0185x over previous
import functools

import jax
import jax.numpy as jnp
from jax import lax
from jax.experimental import pallas as pl
from jax.experimental.pallas import tpu as pltpu
from jax.experimental.pallas import tpu_sc as plsc

VOCAB = 1000000
EMB = 64
NEG = 5
BATCH = 16384

NW = 32
T = 4
S_TOTAL = BATCH * NEG
S_PER_W = S_TOTAL // NW
B_PER_W = BATCH // NW
CHUNK = 128
S_CHUNKS = S_PER_W // CHUNK
B_CHUNKS = B_PER_W // CHUNK


def _rng_body(prop_ref, u4_ref):
    pltpu.prng_seed(42)
    shape = prop_ref.shape
    b1 = pltpu.bitcast(pltpu.prng_random_bits(shape), jnp.uint32)
    b2 = pltpu.bitcast(pltpu.prng_random_bits(shape), jnp.uint32)
    prop_ref[...] = (b1 % jnp.uint32(VOCAB)).astype(jnp.int32)
    u = (b2 >> jnp.uint32(8)).astype(jnp.float32) * jnp.float32(2.0**-24)
    u2 = u * u
    u4_ref[...] = u2 * u2


def _make_rng():
    return pl.pallas_call(
        _rng_body,
        out_shape=(
            jax.ShapeDtypeStruct((NW * T, S_PER_W), jnp.int32),
            jax.ShapeDtypeStruct((NW * T, S_PER_W), jnp.float32),
        ),
    )


def _sc_body(wf_hbm, in_emb_hbm, out_emb_hbm, iw_hbm, cw_hbm, prop_hbm,
             u4_hbm, iv_hbm, ov_hbm, neg_hbm,
             prop_v, u4_v, wf_v, iwcw_v, negidx_v, buf0, buf1, sem_g, sem_w):
    wid = lax.axis_index("s") * 2 + lax.axis_index("c")

    pltpu.sync_copy(prop_hbm.at[pl.ds(wid * T, T)], prop_v)
    pltpu.sync_copy(u4_hbm.at[pl.ds(wid * T, T)], u4_v)
    pltpu.sync_copy(iw_hbm.at[pl.ds(wid * B_PER_W, B_PER_W)], iwcw_v.at[0])
    pltpu.sync_copy(cw_hbm.at[pl.ds(wid * B_PER_W, B_PER_W)], iwcw_v.at[1])

    wf_descs = []
    for t in range(T):
        for c in range(S_CHUNKS):
            sl = pl.ds(c * CHUNK, CHUNK)
            wf_descs.append(pltpu.async_copy(
                wf_hbm.at[prop_v.at[t, sl]], wf_v.at[t, sl], sem_g))

    io_descs = []
    for c in range(B_CHUNKS):
        sl = pl.ds(c * CHUNK, CHUNK)
        io_descs.append(pltpu.async_copy(
            in_emb_hbm.at[iwcw_v.at[0, sl]], buf0.at[sl], sem_g))
        io_descs.append(pltpu.async_copy(
            out_emb_hbm.at[iwcw_v.at[1, sl]], buf1.at[sl], sem_g))
    for d in io_descs:
        d.wait()
    dst = pl.ds(wid * B_PER_W, B_PER_W)
    w_iv = pltpu.async_copy(buf0.at[pl.ds(0, B_PER_W)], iv_hbm.at[dst], sem_w)
    w_ov = pltpu.async_copy(buf1.at[pl.ds(0, B_PER_W)], ov_hbm.at[dst], sem_w)
    for d in wf_descs:
        d.wait()

    def accept(v, carry):
        g = pl.ds(v * 16, 16)
        best = prop_v[T - 1, g]
        for t in reversed(range(T - 1)):
            w = wf_v[t, g]
            ok = u4_v[t, g] <= (w * w) * w
            best = jnp.where(ok, prop_v[t, g], best)
        negidx_v[g] = best
        return carry

    lax.fori_loop(0, S_PER_W // 16, accept, 0)
    w_iv.wait()
    w_ov.wait()

    GROUPS, GC = 4, S_CHUNKS // 4
    GR = GC * CHUNK
    wdescs = [None] * GROUPS
    for g in range(GROUPS):
        buf = buf0 if g % 2 == 0 else buf1
        if g >= 2:
            wdescs[g - 2].wait()
        gds = []
        for c in range(GC):
            cc = g * GC + c
            gds.append(pltpu.async_copy(
                out_emb_hbm.at[negidx_v.at[pl.ds(cc * CHUNK, CHUNK)]],
                buf.at[pl.ds(c * CHUNK, CHUNK)], sem_g))
        for d in gds:
            d.wait()
        wdescs[g] = pltpu.async_copy(
            buf.at[pl.ds(0, GR)],
            neg_hbm.at[pl.ds(wid * S_PER_W + g * GR, GR)], sem_w)
    wdescs[GROUPS - 2].wait()
    wdescs[GROUPS - 1].wait()


def _make_gather():
    mesh = plsc.VectorSubcoreMesh(core_axis_name="c", subcore_axis_name="s")
    return pl.kernel(
        _sc_body,
        mesh=mesh,
        compiler_params=pltpu.CompilerParams(use_tc_tiling_on_sc=False),
        out_type=(
            jax.ShapeDtypeStruct((BATCH, EMB), jnp.float32),
            jax.ShapeDtypeStruct((BATCH, EMB), jnp.float32),
            jax.ShapeDtypeStruct((S_TOTAL, EMB), jnp.float32),
        ),
        scratch_types=[
            pltpu.VMEM((T, S_PER_W), jnp.int32),
            pltpu.VMEM((T, S_PER_W), jnp.float32),
            pltpu.VMEM((T, S_PER_W), jnp.float32),
            pltpu.VMEM((2, B_PER_W), jnp.int32),
            pltpu.VMEM((S_PER_W,), jnp.int32),
            pltpu.VMEM((640, EMB), jnp.float32),
            pltpu.VMEM((640, EMB), jnp.float32),
            pltpu.SemaphoreType.DMA,
            pltpu.SemaphoreType.DMA,
        ],
    )


def _logsig(x):
    return jnp.minimum(x, 0.0) - jnp.log(1.0 + jnp.exp(-jnp.abs(x)))


def _loss_body(iv_ref, ov_ref, neg_ref, out_ref):
    i = pl.program_id(0)
    iv = iv_ref[...]
    pos = jnp.sum(_logsig(iv * ov_ref[...])) * (1.0 / EMB)
    scores = -jnp.sum(neg_ref[...] * iv[None, :, :], axis=2)
    neg = jnp.sum(_logsig(scores))
    @pl.when(i == 0)
    def _():
        out_ref[...] = jnp.zeros((1, 1), jnp.float32)
    out_ref[...] = out_ref[...] + (pos + neg)
    @pl.when(i == NW - 1)
    def _():
        out_ref[...] = out_ref[...] * jnp.float32(-1.0 / BATCH)


def _make_loss():
    return pl.pallas_call(
        _loss_body,
        grid=(NW,),
        in_specs=[
            pl.BlockSpec((B_PER_W, EMB), lambda i: (i, 0)),
            pl.BlockSpec((B_PER_W, EMB), lambda i: (i, 0)),
            pl.BlockSpec((NEG, B_PER_W, EMB), lambda i: (0, i, 0)),
        ],
        out_specs=pl.BlockSpec((1, 1), lambda i: (0, 0)),
        out_shape=jax.ShapeDtypeStruct((1, 1), jnp.float32),
    )


def kernel(input_word, context_word, in_emb, out_emb, word_frequency):
    iw = input_word.astype(jnp.int32)
    cw = context_word.astype(jnp.int32)
    prop, u4 = _make_rng()()
    iv, ov, neg = _make_gather()(word_frequency, in_emb, out_emb, iw, cw,
                                 prop, u4)
    neg3 = neg.reshape(NEG, BATCH, EMB)
    loss = _make_loss()(iv, ov, neg3)
    return loss.reshape(())
